# Initial kernel scaffold; baseline (speedup 1.0000x reference)
import jax
import jax.numpy as jnp
from jax.experimental import pallas as pl


def kernel(pos_enc):
    raise NotImplementedError("write your pallas kernel here")


# speedup vs baseline: 1666.7391x; 1666.7391x over previous
import functools

import jax
import jax.numpy as jnp
from jax import lax
from jax.experimental import pallas as pl
from jax.experimental.pallas import tpu as pltpu
from jax.experimental.pallas import tpu_sc as plsc

NUM_HEADS = 16
H = 32
W = 32
D = 2 * H - 1
TAB = D * D
TAB_PAD = 3976


def _rpe_body(tab_hbm, out_hbm, tab_v, shifted_v, sem):
    c = lax.axis_index("c")
    s = lax.axis_index("s")
    wid = s * 2 + c
    h = wid // 2
    half = wid % 2

    pltpu.sync_copy(tab_hbm.at[h], tab_v)

    def build_x1(x1, carry):
        def build_r(r, carry2):
            b = (62 - r) * D + x1
            v0 = tab_v[pl.ds(b, 16)]
            v1 = tab_v[pl.ds(b + 16, 16)]
            shifted_v[x1, r, pl.ds(0, 16)] = lax.rev(v1, (0,))
            shifted_v[x1, r, pl.ds(16, 16)] = lax.rev(v0, (0,))
            return carry2
        return lax.fori_loop(0, D, build_r, carry)

    lax.fori_loop(0, 32, build_x1, 0)

    copies = []
    for i in range(16):
        y1 = half * 16 + i
        cp = pltpu.make_async_copy(
            shifted_v.at[:, pl.ds(31 - y1, 32), :],
            out_hbm.at[h, y1],
            sem,
        )
        cp.start()
        copies.append(cp)
    for cp in copies:
        cp.wait()


_rpe_kernel = functools.partial(
    pl.kernel,
    mesh=plsc.VectorSubcoreMesh(core_axis_name="c", subcore_axis_name="s"),
    out_type=jax.ShapeDtypeStruct((NUM_HEADS, H, W, H, W), jnp.float32),
    scratch_types=[
        pltpu.VMEM((TAB_PAD,), jnp.float32),
        pltpu.VMEM((W, D, W), jnp.float32),
        pltpu.SemaphoreType.DMA,
    ],
    compiler_params=pltpu.CompilerParams(use_tc_tiling_on_sc=False),
)(_rpe_body)


def kernel(pos_enc):
    tab = jnp.pad(pos_enc, ((0, 0), (0, TAB_PAD - TAB)))
    out = _rpe_kernel(tab)
    return out.reshape(NUM_HEADS, H * W, H * W)
